# baseline (speedup 1.0000x reference)
import functools

import jax
import jax.numpy as jnp
from jax import lax
from jax.experimental import pallas as pl
from jax.experimental.pallas import tpu as pltpu
from jax.experimental.pallas import tpu_sc as plsc

_NUM_CORES = 2
_NUM_SUBCORES = 16
_NUM_WORKERS = _NUM_CORES * _NUM_SUBCORES

_CHUNK = 512


@functools.lru_cache(maxsize=None)
def _make_gather(n, v, d):
    n_per_w = n // _NUM_WORKERS
    n_chunks = n_per_w // _CHUNK
    assert n_per_w * _NUM_WORKERS == n and n_chunks * _CHUNK == n_per_w

    mesh = plsc.VectorSubcoreMesh(
        core_axis_name="c", subcore_axis_name="s", num_cores=_NUM_CORES
    )

    @functools.partial(
        pl.kernel,
        out_type=jax.ShapeDtypeStruct((n, d), jnp.float32),
        mesh=mesh,
        scratch_types=[
            pltpu.VMEM((_CHUNK,), jnp.int32),
            pltpu.VMEM((_CHUNK, d), jnp.float32),
            pltpu.SemaphoreType.DMA,
        ],
        compiler_params=pltpu.CompilerParams(use_tc_tiling_on_sc=False),
    )
    def gather_kernel(idx_hbm, table_hbm, out_hbm, idx_v, rows_v, sem):
        wid = lax.axis_index("s") * _NUM_CORES + lax.axis_index("c")
        base = wid * n_per_w

        def body(g, carry):
            off = base + g * _CHUNK
            pltpu.sync_copy(idx_hbm.at[pl.ds(off, _CHUNK)], idx_v)
            pltpu.async_copy(table_hbm.at[idx_v], rows_v, sem).wait()
            pltpu.sync_copy(rows_v, out_hbm.at[pl.ds(off, _CHUNK)])
            return carry

        lax.fori_loop(0, n_chunks, body, 0)

    return gather_kernel


def kernel(tokens, table):
    b, s = tokens.shape
    v, d = table.shape
    n = b * s
    idx = tokens.reshape(n).astype(jnp.int32)
    out = _make_gather(n, v, d)(idx, table)
    return out.reshape(b, s, d)


# speedup vs baseline: 1.0410x; 1.0410x over previous
import functools

import jax
import jax.numpy as jnp
from jax import lax
from jax.experimental import pallas as pl
from jax.experimental.pallas import tpu as pltpu
from jax.experimental.pallas import tpu_sc as plsc

_NUM_CORES = 2
_NUM_SUBCORES = 16
_NUM_WORKERS = _NUM_CORES * _NUM_SUBCORES

_CHUNK = 512
_NBUF = 2


@functools.lru_cache(maxsize=None)
def _make_gather(n, v, d):
    n_per_w = n // _NUM_WORKERS
    n_chunks = n_per_w // _CHUNK
    assert n_per_w * _NUM_WORKERS == n and n_chunks * _CHUNK == n_per_w
    assert n_chunks % _NBUF == 0

    mesh = plsc.VectorSubcoreMesh(
        core_axis_name="c", subcore_axis_name="s", num_cores=_NUM_CORES
    )

    @functools.partial(
        pl.kernel,
        out_type=jax.ShapeDtypeStruct((n, d), jnp.float32),
        mesh=mesh,
        scratch_types=[
            pltpu.VMEM((n_per_w,), jnp.int32),
            tuple(pltpu.VMEM((_CHUNK, d), jnp.float32) for _ in range(_NBUF)),
            tuple(pltpu.SemaphoreType.DMA for _ in range(_NBUF)),
            tuple(pltpu.SemaphoreType.DMA for _ in range(_NBUF)),
        ],
        compiler_params=pltpu.CompilerParams(use_tc_tiling_on_sc=False),
    )
    def gather_kernel(idx_hbm, table_hbm, out_hbm, idx_all, rows, gsems, wsems):
        wid = lax.axis_index("s") * _NUM_CORES + lax.axis_index("c")
        base = wid * n_per_w

        pltpu.sync_copy(idx_hbm.at[pl.ds(base, n_per_w)], idx_all)

        def g_start(g, b):
            pltpu.async_copy(
                table_hbm.at[idx_all.at[pl.ds(g * _CHUNK, _CHUNK)]],
                rows[b], gsems[b])

        def g_wait(b):
            pltpu.make_async_copy(
                table_hbm.at[idx_all.at[pl.ds(0, _CHUNK)]],
                rows[b], gsems[b]).wait()

        def w_start(g, b):
            pltpu.async_copy(
                rows[b], out_hbm.at[pl.ds(base + g * _CHUNK, _CHUNK)],
                wsems[b])

        def w_wait(b):
            pltpu.make_async_copy(
                rows[b], out_hbm.at[pl.ds(base, _CHUNK)], wsems[b]).wait()

        for j in range(_NBUF - 1):
            g_start(j, j)

        def step(g, b):
            g_wait(b)
            pb = (b - 1) % _NBUF

            @pl.when(g >= 1)
            def _():
                w_wait(pb)

            @pl.when(g + _NBUF - 1 < n_chunks)
            def _():
                g_start(g + _NBUF - 1, pb)

            w_start(g, b)

        def outer(i, carry):
            for b in range(_NBUF):
                step(i * _NBUF + b, b)
            return carry

        lax.fori_loop(0, n_chunks // _NBUF, outer, 0)
        w_wait((n_chunks - 1) % _NBUF)

    return gather_kernel


def kernel(tokens, table):
    b, s = tokens.shape
    v, d = table.shape
    n = b * s
    idx = tokens.reshape(n).astype(jnp.int32)
    out = _make_gather(n, v, d)(idx, table)
    return out.reshape(b, s, d)
